# baseline (speedup 1.0000x reference)
import functools

import jax
import jax.numpy as jnp
from jax import lax
from jax.experimental import pallas as pl
from jax.experimental.pallas import tpu as pltpu
from jax.experimental.pallas import tpu_sc as plsc

N = 10000
E = 320000
D = 128
NPAD = 10240
NC, NS = 2, 16
NW = NC * NS
KB = 128
EROWS = 2560
E_PAD = EROWS * KB
NBUF = 2
RF = 80
RS = 80
HF = HS = RF // 2
DEG_ROWS = EROWS // NS
DEG_SLICE = NPAD // NS


def _fill_1d(ref, n, value, dtype):
    def body(i, carry):
        ref[pl.ds(i * 16, 16)] = jnp.full((16,), value, dtype)
        return carry
    lax.fori_loop(0, n // 16, body, 0)


@functools.partial(
    pl.kernel,
    out_type=jax.ShapeDtypeStruct((NC * NPAD,), jnp.float32),
    mesh=plsc.VectorSubcoreMesh(core_axis_name="c", subcore_axis_name="s"),
    scratch_types=[
        pltpu.VMEM_SHARED((NPAD,), jnp.float32),
        pltpu.VMEM((DEG_SLICE,), jnp.float32),
        pltpu.VMEM((KB,), jnp.float32),
        pltpu.VMEM((DEG_ROWS, KB), jnp.int32),
        pltpu.SemaphoreType.DMA,
    ],
)
def _deg_kernel(src_hbm, dst_hbm, out_hbm, degbuf, zbuf, ones, idxbuf, ssem):
    c = lax.axis_index("c")
    s = lax.axis_index("s")
    _fill_1d(zbuf, DEG_SLICE, 0.0, jnp.float32)
    pltpu.sync_copy(zbuf, degbuf.at[pl.ds(s * DEG_SLICE, DEG_SLICE)])
    _fill_1d(ones, KB, 1.0, jnp.float32)

    base = s * DEG_ROWS
    @pl.when(c == 0)
    def _():
        pltpu.sync_copy(src_hbm.at[pl.ds(base, DEG_ROWS)], idxbuf)

    @pl.when(c == 1)
    def _():
        pltpu.sync_copy(dst_hbm.at[pl.ds(base, DEG_ROWS)], idxbuf)

    plsc.subcore_barrier()

    def fire(g, carry):
        def body(j, carry2):
            pltpu.async_copy(ones, degbuf.at[idxbuf.at[g * 16 + j]], ssem,
                             add=True)
            return carry2
        lax.fori_loop(0, 16, body, 0)
        def drain(j, carry2):
            pltpu.make_async_copy(ones, degbuf.at[idxbuf.at[0]], ssem).wait()
            return carry2
        lax.fori_loop(0, 16, drain, 0)
        return carry
    lax.fori_loop(0, DEG_ROWS // 16, fire, 0)

    plsc.subcore_barrier()
    pltpu.sync_copy(degbuf.at[pl.ds(s * DEG_SLICE, DEG_SLICE)],
                    out_hbm.at[pl.ds(c * NPAD + s * DEG_SLICE, DEG_SLICE)])


@functools.partial(
    pl.kernel,
    out_type=jax.ShapeDtypeStruct((NC, NPAD, D), jnp.float32),
    mesh=plsc.VectorSubcoreMesh(core_axis_name="c", subcore_axis_name="s"),
    scratch_types=[
        pltpu.VMEM_SHARED((NPAD, D), jnp.float32),
        pltpu.VMEM((HF, KB), jnp.int32),
        pltpu.VMEM((HF, KB), jnp.int32),
        pltpu.VMEM((KB, D), jnp.float32),
        pltpu.VMEM((KB, D), jnp.float32),
        pltpu.SemaphoreType.DMA,
        pltpu.SemaphoreType.DMA,
        pltpu.SemaphoreType.DMA,
        pltpu.SemaphoreType.DMA,
    ],
)
def _agg_kernel(y_hbm, src_hbm, dst_hbm, out_hbm, aggbuf, sidx, didx,
                r0, r1, g0, g1, s0, s1):
    c = lax.axis_index("c")
    s = lax.axis_index("s")
    rows = (r0, r1)
    gsem = (g0, g1)
    ssem = (s0, s1)

    def zb(i, carry):
        r0[i // 8, pl.ds((i % 8) * 16, 16)] = jnp.zeros((16,), jnp.float32)
        return carry
    lax.fori_loop(0, KB * D // 16, zb, 0)
    for t in range(DEG_SLICE // KB):
        pltpu.sync_copy(r0, aggbuf.at[pl.ds(s * DEG_SLICE + t * KB, KB)])
    plsc.subcore_barrier()

    def run(base, half):
        groups = half // NBUF
        for h in range(2):
            pltpu.sync_copy(src_hbm.at[pl.ds(base + h * half, half)],
                            sidx.at[pl.ds(0, half)])
            pltpu.sync_copy(dst_hbm.at[pl.ds(base + h * half, half)],
                            didx.at[pl.ds(0, half)])

            for b in range(NBUF):
                pltpu.async_copy(y_hbm.at[sidx.at[b]], rows[b], gsem[b])

            def group(gi, carry):
                for b in range(NBUF):
                    j = gi * NBUF + b
                    pltpu.make_async_copy(y_hbm.at[sidx.at[j]], rows[b],
                                          gsem[b]).wait()
                    pltpu.async_copy(rows[b], aggbuf.at[didx.at[j]],
                                     ssem[b], add=True)
                for b in range(NBUF):
                    j = gi * NBUF + b
                    pltpu.make_async_copy(rows[b], aggbuf.at[didx.at[j]],
                                          ssem[b]).wait()
                    pltpu.async_copy(y_hbm.at[sidx.at[j + NBUF]], rows[b],
                                     gsem[b])
                return carry
            lax.fori_loop(0, groups - 1, group, 0)

            jf = (groups - 1) * NBUF
            for b in range(NBUF):
                pltpu.make_async_copy(y_hbm.at[sidx.at[jf + b]], rows[b],
                                      gsem[b]).wait()
                pltpu.async_copy(rows[b], aggbuf.at[didx.at[jf + b]],
                                 ssem[b], add=True)
            for b in range(NBUF):
                pltpu.make_async_copy(rows[b], aggbuf.at[didx.at[jf + b]],
                                      ssem[b]).wait()

    run((c * NS + s) * RF, HF)

    plsc.subcore_barrier()
    pltpu.sync_copy(aggbuf.at[pl.ds(s * DEG_SLICE, DEG_SLICE)],
                    out_hbm.at[c, pl.ds(s * DEG_SLICE, DEG_SLICE)])


def _scale_body(x_ref, d_ref, y_ref):
    a = lax.rsqrt(jnp.maximum(d_ref[...], 1.0))
    y_ref[...] = x_ref[...] * a


_scale = pl.pallas_call(
    _scale_body,
    grid=(10,),
    in_specs=[
        pl.BlockSpec((N // 10, D), lambda i: (i, 0)),
        pl.BlockSpec((N // 10, 1), lambda i: (i, 0)),
    ],
    out_specs=pl.BlockSpec((N // 10, D), lambda i: (i, 0)),
    out_shape=jax.ShapeDtypeStruct((NPAD, D), jnp.float32),
)


def _mlp_body(p0_ref, p1_ref, d_ref, w1t_ref, b1_ref, w2t_ref, b2_ref, o_ref):
    binv = lax.rsqrt(jnp.maximum(d_ref[...], 1.0))
    agg = (p0_ref[...] + p1_ref[...]) * binv
    h = jnp.dot(agg, w1t_ref[...], preferred_element_type=jnp.float32)
    h = h + b1_ref[...]
    h = 0.5 * h * (1.0 + lax.erf(h * 0.7071067811865476))
    o_ref[...] = (jnp.dot(h, w2t_ref[...], preferred_element_type=jnp.float32)
                  + b2_ref[...])


_mlp = pl.pallas_call(
    _mlp_body,
    grid=(10,),
    in_specs=[
        pl.BlockSpec((NPAD // 10, D), lambda i: (i, 0)),
        pl.BlockSpec((NPAD // 10, D), lambda i: (i, 0)),
        pl.BlockSpec((NPAD // 10, 1), lambda i: (i, 0)),
        pl.BlockSpec((D, D), lambda i: (0, 0)),
        pl.BlockSpec((1, D), lambda i: (0, 0)),
        pl.BlockSpec((D, D), lambda i: (0, 0)),
        pl.BlockSpec((1, D), lambda i: (0, 0)),
    ],
    out_specs=pl.BlockSpec((NPAD // 10, D), lambda i: (i, 0)),
    out_shape=jax.ShapeDtypeStruct((NPAD, D), jnp.float32),
)


def kernel(x, edge_index, W1, b1, W2, b2):
    pad = jnp.full((E_PAD - E,), N, jnp.int32)
    src = jnp.concatenate([edge_index[0].astype(jnp.int32), pad])
    dst = jnp.concatenate([edge_index[1].astype(jnp.int32), pad])
    src = src.reshape(EROWS, KB)
    dst = dst.reshape(EROWS, KB)
    deg = _deg_kernel(src, dst).reshape(NC, NPAD)
    outdeg = deg[0, :N].reshape(N, 1)
    indeg = deg[1].reshape(NPAD, 1)
    y = _scale(x, outdeg)
    partial = _agg_kernel(y, src, dst)
    out = _mlp(partial[0], partial[1], indeg,
               W1.T, b1.reshape(1, D), W2.T, b2.reshape(1, D))
    return out[:N]


# speedup vs baseline: 1.1346x; 1.1346x over previous
import functools

import jax
import jax.numpy as jnp
from jax import lax
from jax.experimental import pallas as pl
from jax.experimental.pallas import tpu as pltpu
from jax.experimental.pallas import tpu_sc as plsc

N = 10000
E = 320000
D = 128
NPAD = 10240
NC, NS = 2, 16
NW = NC * NS
KB = 128
EROWS = 2560
E_PAD = EROWS * KB
NBUF = 2
RF = 80
RS = 80
HF = HS = RF // 2
DEG_ROWS = EROWS // NS
DEG_SLICE = NPAD // NS


def _fill_1d(ref, n, value, dtype):
    def body(i, carry):
        ref[pl.ds(i * 16, 16)] = jnp.full((16,), value, dtype)
        return carry
    lax.fori_loop(0, n // 16, body, 0)


@functools.partial(
    pl.kernel,
    out_type=jax.ShapeDtypeStruct((NC * NPAD,), jnp.float32),
    mesh=plsc.VectorSubcoreMesh(core_axis_name="c", subcore_axis_name="s"),
    scratch_types=[
        pltpu.VMEM_SHARED((NPAD,), jnp.float32),
        pltpu.VMEM((DEG_SLICE,), jnp.float32),
        pltpu.VMEM((KB,), jnp.float32),
        pltpu.VMEM((DEG_ROWS, KB), jnp.int32),
        pltpu.SemaphoreType.DMA,
    ],
)
def _deg_kernel(src_hbm, dst_hbm, out_hbm, degbuf, zbuf, ones, idxbuf, ssem):
    c = lax.axis_index("c")
    s = lax.axis_index("s")
    _fill_1d(zbuf, DEG_SLICE, 0.0, jnp.float32)
    pltpu.sync_copy(zbuf, degbuf.at[pl.ds(s * DEG_SLICE, DEG_SLICE)])
    _fill_1d(ones, KB, 1.0, jnp.float32)

    base = s * DEG_ROWS
    @pl.when(c == 0)
    def _():
        pltpu.sync_copy(src_hbm.at[pl.ds(base, DEG_ROWS)], idxbuf)

    @pl.when(c == 1)
    def _():
        pltpu.sync_copy(dst_hbm.at[pl.ds(base, DEG_ROWS)], idxbuf)

    plsc.subcore_barrier()

    def fire(g, carry):
        def body(j, carry2):
            pltpu.async_copy(ones, degbuf.at[idxbuf.at[g * 16 + j]], ssem,
                             add=True)
            return carry2
        lax.fori_loop(0, 16, body, 0)
        def drain(j, carry2):
            pltpu.make_async_copy(ones, degbuf.at[idxbuf.at[0]], ssem).wait()
            return carry2
        lax.fori_loop(0, 16, drain, 0)
        return carry
    lax.fori_loop(0, DEG_ROWS // 16, fire, 0)

    plsc.subcore_barrier()
    pltpu.sync_copy(degbuf.at[pl.ds(s * DEG_SLICE, DEG_SLICE)],
                    out_hbm.at[pl.ds(c * NPAD + s * DEG_SLICE, DEG_SLICE)])


@functools.partial(
    pl.kernel,
    out_type=jax.ShapeDtypeStruct((NC, NPAD, D), jnp.float32),
    mesh=plsc.VectorSubcoreMesh(core_axis_name="c", subcore_axis_name="s"),
    scratch_types=[
        pltpu.VMEM_SHARED((NPAD, D), jnp.float32),
        pltpu.VMEM((HF, KB), jnp.int32),
        pltpu.VMEM((HF, KB), jnp.int32),
        pltpu.VMEM((KB, D), jnp.float32),
        pltpu.VMEM((KB, D), jnp.float32),
        pltpu.SemaphoreType.DMA,
        pltpu.SemaphoreType.DMA,
        pltpu.SemaphoreType.DMA,
        pltpu.SemaphoreType.DMA,
    ],
)
def _agg_kernel(y_hbm, src_hbm, dst_hbm, out_hbm, aggbuf, sidx, didx,
                r0, r1, g0, g1, s0, s1):
    c = lax.axis_index("c")
    s = lax.axis_index("s")
    rows = (r0, r1)
    gsem = (g0, g1)
    ssem = (s0, s1)

    def zb(i, carry):
        r0[i // 8, pl.ds((i % 8) * 16, 16)] = jnp.zeros((16,), jnp.float32)
        return carry
    lax.fori_loop(0, KB * D // 16, zb, 0)
    for t in range(DEG_SLICE // KB):
        pltpu.sync_copy(r0, aggbuf.at[pl.ds(s * DEG_SLICE + t * KB, KB)])
    plsc.subcore_barrier()

    def run(base, half):
        groups = half // NBUF
        for h in range(2):
            pltpu.sync_copy(src_hbm.at[pl.ds(base + h * half, half)],
                            sidx.at[pl.ds(0, half)])
            pltpu.sync_copy(dst_hbm.at[pl.ds(base + h * half, half)],
                            didx.at[pl.ds(0, half)])

            for b in range(NBUF):
                pltpu.async_copy(y_hbm.at[sidx.at[b]], rows[b], gsem[b])

            def group(gi, carry):
                for b in range(NBUF):
                    j = gi * NBUF + b
                    pltpu.make_async_copy(y_hbm.at[sidx.at[j]], rows[b],
                                          gsem[b]).wait()
                    pltpu.async_copy(rows[b], aggbuf.at[didx.at[j]],
                                     ssem[b], add=True)
                for b in range(NBUF):
                    j = gi * NBUF + b
                    pltpu.make_async_copy(rows[b], aggbuf.at[didx.at[j]],
                                          ssem[b]).wait()
                    pltpu.async_copy(y_hbm.at[sidx.at[j + NBUF]], rows[b],
                                     gsem[b])
                return carry
            lax.fori_loop(0, groups - 1, group, 0)

            jf = (groups - 1) * NBUF
            for b in range(NBUF):
                pltpu.make_async_copy(y_hbm.at[sidx.at[jf + b]], rows[b],
                                      gsem[b]).wait()
                pltpu.async_copy(rows[b], aggbuf.at[didx.at[jf + b]],
                                 ssem[b], add=True)
            for b in range(NBUF):
                pltpu.make_async_copy(rows[b], aggbuf.at[didx.at[jf + b]],
                                      ssem[b]).wait()

    run((c * NS + s) * RF, HF)

    plsc.subcore_barrier()
    pltpu.sync_copy(aggbuf.at[pl.ds(s * DEG_SLICE, DEG_SLICE)],
                    out_hbm.at[c, pl.ds(s * DEG_SLICE, DEG_SLICE)])


def _scale_body(x_ref, d_ref, y_ref):
    a = lax.rsqrt(jnp.maximum(d_ref[...], 1.0))
    y_ref[...] = x_ref[...] * a


_scale = pl.pallas_call(
    _scale_body,
    grid=(10,),
    in_specs=[
        pl.BlockSpec((NPAD // 10, D), lambda i: (i, 0)),
        pl.BlockSpec((NPAD // 10, 1), lambda i: (i, 0)),
    ],
    out_specs=pl.BlockSpec((NPAD // 10, D), lambda i: (i, 0)),
    out_shape=jax.ShapeDtypeStruct((NPAD, D), jnp.float32),
)


def _mlp_body(p0_ref, p1_ref, d_ref, w1t_ref, b1_ref, w2t_ref, b2_ref, o_ref):
    binv = lax.rsqrt(jnp.maximum(d_ref[...], 1.0))
    agg = (p0_ref[...] + p1_ref[...]) * binv
    h = jnp.dot(agg, w1t_ref[...], preferred_element_type=jnp.float32)
    h = h + b1_ref[...]
    h = 0.5 * h * (1.0 + lax.erf(h * 0.7071067811865476))
    o_ref[...] = (jnp.dot(h, w2t_ref[...], preferred_element_type=jnp.float32)
                  + b2_ref[...])


_mlp = pl.pallas_call(
    _mlp_body,
    grid=(10,),
    in_specs=[
        pl.BlockSpec((NPAD // 10, D), lambda i: (i, 0)),
        pl.BlockSpec((NPAD // 10, D), lambda i: (i, 0)),
        pl.BlockSpec((NPAD // 10, 1), lambda i: (i, 0)),
        pl.BlockSpec((D, D), lambda i: (0, 0)),
        pl.BlockSpec((1, D), lambda i: (0, 0)),
        pl.BlockSpec((D, D), lambda i: (0, 0)),
        pl.BlockSpec((1, D), lambda i: (0, 0)),
    ],
    out_specs=pl.BlockSpec((NPAD // 10, D), lambda i: (i, 0)),
    out_shape=jax.ShapeDtypeStruct((NPAD, D), jnp.float32),
)


def kernel(x, edge_index, W1, b1, W2, b2):
    pad = jnp.full((E_PAD - E,), N, jnp.int32)
    src = jnp.concatenate([edge_index[0].astype(jnp.int32), pad])
    dst = jnp.concatenate([edge_index[1].astype(jnp.int32), pad])
    src = src.reshape(EROWS, KB)
    dst = dst.reshape(EROWS, KB)
    x_pad = jnp.pad(x, ((0, NPAD - N), (0, 0)))
    deg = _deg_kernel(src, dst).reshape(NC, NPAD)
    outdeg = deg[0].reshape(NPAD, 1)
    indeg = deg[1].reshape(NPAD, 1)
    y = _scale(x_pad, outdeg)
    partial = _agg_kernel(y, src, dst)
    out = _mlp(partial[0], partial[1], indeg,
               W1.T, b1.reshape(1, D), W2.T, b2.reshape(1, D))
    return out[:N]
